# baseline (speedup 1.0000x reference)
import functools

import jax
import jax.numpy as jnp
from jax import lax
from jax.experimental import pallas as pl
from jax.experimental.pallas import tpu as pltpu
from jax.experimental.pallas import tpu_sc as plsc

N = 10000
E = 320000
D = 128
EPS = 1e-5

NC = 2
NS = 16
NW = NC * NS
K = 128
CHUNKS = 80
EPAD = NW * CHUNKS * K
NPAD = 10240
STRIPE = NPAD // NS

_mesh = plsc.VectorSubcoreMesh(core_axis_name="c", subcore_axis_name="s",
                               num_cores=NC, num_subcores=NS)


@functools.partial(
    pl.kernel,
    out_type=jax.ShapeDtypeStruct((NC, 2, NPAD), jnp.float32),
    mesh=_mesh,
    scratch_types=[
        pltpu.VMEM((CHUNKS, K), jnp.int32),
        pltpu.VMEM((CHUNKS, K), jnp.int32),
        pltpu.VMEM((K,), jnp.float32),
        pltpu.VMEM_SHARED((NPAD,), jnp.float32),
        pltpu.VMEM_SHARED((NPAD,), jnp.float32),
    ],
)
def _sc_degrees(src_hbm, dst_hbm, zrow_hbm, out_hbm, sbuf, dbuf, ones_v,
                outd_s, ind_s):
    c = lax.axis_index("c")
    s = lax.axis_index("s")
    wid = c * NS + s
    for i in range(K // 16):
        ones_v[pl.ds(i * 16, 16)] = jnp.ones((16,), jnp.float32)
    rows = pl.ds(s * STRIPE, STRIPE)
    pltpu.sync_copy(zrow_hbm.at[pl.ds(0, STRIPE)], outd_s.at[rows])
    pltpu.sync_copy(zrow_hbm.at[pl.ds(0, STRIPE)], ind_s.at[rows])
    pltpu.sync_copy(src_hbm.at[pl.ds(wid * CHUNKS, CHUNKS), :], sbuf)
    pltpu.sync_copy(dst_hbm.at[pl.ds(wid * CHUNKS, CHUNKS), :], dbuf)
    plsc.subcore_barrier()

    @pl.loop(0, CHUNKS)
    def _(i):
        pltpu.sync_copy(ones_v, outd_s.at[sbuf.at[i]], add=True)
        pltpu.sync_copy(ones_v, ind_s.at[dbuf.at[i]], add=True)

    plsc.subcore_barrier()
    pltpu.sync_copy(outd_s.at[rows], out_hbm.at[c, 0, rows])
    pltpu.sync_copy(ind_s.at[rows], out_hbm.at[c, 1, rows])


NB = 2
CH0 = 32
CH1 = 128
NSTAGE = 4
QMAX = max(CH0, CH1) // NSTAGE


@functools.partial(
    pl.kernel,
    out_type=jax.ShapeDtypeStruct((NC, NPAD, D), jnp.float32),
    mesh=_mesh,
    scratch_types=[
        pltpu.VMEM((QMAX, K), jnp.int32),
        pltpu.VMEM((QMAX, K), jnp.int32),
        pltpu.VMEM((NB, K, D), jnp.float32),
        pltpu.VMEM_SHARED((NPAD, D), jnp.float32),
        pltpu.SemaphoreType.DMA((NB,)),
    ],
)
def _sc_aggregate(t_hbm, src_hbm, dst_hbm, zmat_hbm, out_hbm, sbuf, dbuf,
                  rows_v, acc_s, gsem):
    c = lax.axis_index("c")
    s = lax.axis_index("s")
    rows = pl.ds(s * STRIPE, STRIPE)
    pltpu.sync_copy(zmat_hbm.at[rows], acc_s.at[rows])
    plsc.subcore_barrier()

    my_ch = jnp.where(c == 0, CH0, CH1)
    quarter = my_ch // NSTAGE
    tile_base = jnp.where(c == 0, s * CH0, NS * CH0 + s * CH1)

    @pl.loop(0, NSTAGE)
    def _(h):
        cbase = pl.multiple_of(tile_base + h * quarter, 8)
        pltpu.sync_copy(src_hbm.at[pl.ds(cbase, QMAX), :], sbuf)
        pltpu.sync_copy(dst_hbm.at[pl.ds(cbase, QMAX), :], dbuf)
        for b in range(NB):
            pltpu.async_copy(t_hbm.at[sbuf.at[b]], rows_v.at[b], gsem.at[b])

        @pl.loop(0, quarter // NB - 1)
        def _(g):
            for b in range(NB):
                i = g * NB + b
                pltpu.make_async_copy(t_hbm.at[sbuf.at[i]], rows_v.at[b],
                                      gsem.at[b]).wait()
                pltpu.sync_copy(rows_v.at[b], acc_s.at[dbuf.at[i]], add=True)
                pltpu.async_copy(t_hbm.at[sbuf.at[i + NB]], rows_v.at[b],
                                 gsem.at[b])

        for b in range(NB):
            i = quarter - NB + b
            pltpu.make_async_copy(t_hbm.at[sbuf.at[i]], rows_v.at[b],
                                  gsem.at[b]).wait()
            pltpu.sync_copy(rows_v.at[b], acc_s.at[dbuf.at[i]], add=True)

    plsc.subcore_barrier()
    pltpu.sync_copy(acc_s.at[rows], out_hbm.at[c, rows])


R = 512
GRID = NPAD // R


def _norms(deg_blk):
    out_deg = deg_blk[:, 0:1] + deg_blk[:, 2:3]
    in_deg = deg_blk[:, 1:2] + deg_blk[:, 3:4]
    out_norm = lax.rsqrt(jnp.maximum(out_deg, 1.0))
    in_norm = lax.rsqrt(jnp.maximum(in_deg, 1.0))
    return out_norm, in_norm


def _layer_norm_relu(x, g, be):
    mu = jnp.mean(x, axis=-1, keepdims=True)
    var = jnp.mean((x - mu) ** 2, axis=-1, keepdims=True)
    return jnp.maximum((x - mu) * lax.rsqrt(var + EPS) * g + be, 0.0)


def _tc1_body(f_ref, deg_ref, w_ref, t_ref):
    out_norm, _ = _norms(deg_ref[...])
    t_ref[...] = jnp.dot(f_ref[...] * out_norm, w_ref[...],
                         preferred_element_type=jnp.float32)


def _tc2_body(agg_ref, deg_ref, b_ref, g_ref, be_ref, w_ref, t_ref):
    out_norm, in_norm = _norms(deg_ref[...])
    x = (agg_ref[0] + agg_ref[1]) * in_norm + b_ref[...]
    h = _layer_norm_relu(x, g_ref[...], be_ref[...])
    t_ref[...] = jnp.dot(h * out_norm, w_ref[...],
                         preferred_element_type=jnp.float32)


def _tc3_body(agg_ref, deg_ref, b_ref, g_ref, be_ref, o_ref):
    _, in_norm = _norms(deg_ref[...])
    x = (agg_ref[0] + agg_ref[1]) * in_norm + b_ref[...]
    o_ref[...] = _layer_norm_relu(x, g_ref[...], be_ref[...])


_row_spec = pl.BlockSpec((R, D), lambda i: (i, 0))
_deg_spec = pl.BlockSpec((R, 4), lambda i: (i, 0))
_agg_spec = pl.BlockSpec((NC, R, D), lambda i: (0, i, 0))
_w_spec = pl.BlockSpec((D, D), lambda i: (0, 0))
_vec_spec = pl.BlockSpec((1, D), lambda i: (0, 0))

_tc1 = pl.pallas_call(
    _tc1_body,
    grid=(GRID,),
    in_specs=[_row_spec, _deg_spec, _w_spec],
    out_specs=_row_spec,
    out_shape=jax.ShapeDtypeStruct((NPAD, D), jnp.float32),
)

_tc2 = pl.pallas_call(
    _tc2_body,
    grid=(GRID,),
    in_specs=[_agg_spec, _deg_spec, _vec_spec, _vec_spec, _vec_spec, _w_spec],
    out_specs=_row_spec,
    out_shape=jax.ShapeDtypeStruct((NPAD, D), jnp.float32),
)

_tc3 = pl.pallas_call(
    _tc3_body,
    grid=(GRID,),
    in_specs=[_agg_spec, _deg_spec, _vec_spec, _vec_spec, _vec_spec],
    out_specs=_row_spec,
    out_shape=jax.ShapeDtypeStruct((NPAD, D), jnp.float32),
)


def kernel(features, edge_index, W1, b1, g1, be1, W2, b2, g2, be2):
    src = edge_index[0]
    dst = edge_index[1]
    pad = jnp.full((EPAD + QMAX * K - E,), N, dtype=jnp.int32)
    src2d = jnp.concatenate([src, pad]).reshape(EPAD // K + QMAX, K)
    dst2d = jnp.concatenate([dst, pad]).reshape(EPAD // K + QMAX, K)
    fpad = jnp.pad(features, ((0, NPAD - N), (0, 0)))
    zrow = jnp.zeros((NPAD,), jnp.float32)
    zmat = jnp.zeros((NPAD, D), jnp.float32)

    degp = _sc_degrees(src2d, dst2d, zrow)
    degs = jnp.moveaxis(degp.reshape(4, NPAD), 0, 1)

    t1 = _tc1(fpad, degs, W1)
    agg1 = _sc_aggregate(t1, src2d, dst2d, zmat)
    t2 = _tc2(agg1, degs, b1.reshape(1, D), g1.reshape(1, D),
              be1.reshape(1, D), W2)
    agg2 = _sc_aggregate(t2, src2d, dst2d, zmat)
    out = _tc3(agg2, degs, b2.reshape(1, D), g2.reshape(1, D),
               be2.reshape(1, D))
    return out[:N]


# speedup vs baseline: 1.1173x; 1.1173x over previous
import functools

import jax
import jax.numpy as jnp
from jax import lax
from jax.experimental import pallas as pl
from jax.experimental.pallas import tpu as pltpu
from jax.experimental.pallas import tpu_sc as plsc

N = 10000
E = 320000
D = 128
EPS = 1e-5

NC = 2
NS = 16
NW = NC * NS
K = 128
CHUNKS = 80
EPAD = NW * CHUNKS * K
NPAD = 10240
STRIPE = NPAD // NS

_mesh = plsc.VectorSubcoreMesh(core_axis_name="c", subcore_axis_name="s",
                               num_cores=NC, num_subcores=NS)


@functools.partial(
    pl.kernel,
    out_type=jax.ShapeDtypeStruct((NC, 2, NPAD), jnp.float32),
    mesh=_mesh,
    scratch_types=[
        pltpu.VMEM((CHUNKS, K), jnp.int32),
        pltpu.VMEM((CHUNKS, K), jnp.int32),
        pltpu.VMEM((K,), jnp.float32),
        pltpu.VMEM_SHARED((NPAD,), jnp.float32),
        pltpu.VMEM_SHARED((NPAD,), jnp.float32),
    ],
)
def _sc_degrees(src_hbm, dst_hbm, zrow_hbm, out_hbm, sbuf, dbuf, ones_v,
                outd_s, ind_s):
    c = lax.axis_index("c")
    s = lax.axis_index("s")
    wid = c * NS + s
    for i in range(K // 16):
        ones_v[pl.ds(i * 16, 16)] = jnp.ones((16,), jnp.float32)
    rows = pl.ds(s * STRIPE, STRIPE)
    pltpu.sync_copy(zrow_hbm.at[pl.ds(0, STRIPE)], outd_s.at[rows])
    pltpu.sync_copy(zrow_hbm.at[pl.ds(0, STRIPE)], ind_s.at[rows])
    pltpu.sync_copy(src_hbm.at[pl.ds(wid * CHUNKS, CHUNKS), :], sbuf)
    pltpu.sync_copy(dst_hbm.at[pl.ds(wid * CHUNKS, CHUNKS), :], dbuf)
    plsc.subcore_barrier()

    @pl.loop(0, CHUNKS)
    def _(i):
        pltpu.sync_copy(ones_v, outd_s.at[sbuf.at[i]], add=True)
        pltpu.sync_copy(ones_v, ind_s.at[dbuf.at[i]], add=True)

    plsc.subcore_barrier()
    pltpu.sync_copy(outd_s.at[rows], out_hbm.at[c, 0, rows])
    pltpu.sync_copy(ind_s.at[rows], out_hbm.at[c, 1, rows])


NB = 2
CH0 = 128
CH1 = 32
NSTAGE = 4
QMAX = max(CH0, CH1) // NSTAGE


@functools.partial(
    pl.kernel,
    out_type=jax.ShapeDtypeStruct((NC, NPAD, D), jnp.float32),
    mesh=_mesh,
    scratch_types=[
        pltpu.VMEM((QMAX, K), jnp.int32),
        pltpu.VMEM((QMAX, K), jnp.int32),
        pltpu.VMEM((NB, K, D), jnp.float32),
        pltpu.VMEM_SHARED((NPAD, D), jnp.float32),
        pltpu.SemaphoreType.DMA((NB,)),
    ],
)
def _sc_aggregate(t_hbm, src_hbm, dst_hbm, zmat_hbm, out_hbm, sbuf, dbuf,
                  rows_v, acc_s, gsem):
    c = lax.axis_index("c")
    s = lax.axis_index("s")
    rows = pl.ds(s * STRIPE, STRIPE)
    pltpu.sync_copy(zmat_hbm.at[rows], acc_s.at[rows])
    plsc.subcore_barrier()

    my_ch = jnp.where(c == 0, CH0, CH1)
    quarter = my_ch // NSTAGE
    tile_base = jnp.where(c == 0, s * CH0, NS * CH0 + s * CH1)

    @pl.loop(0, NSTAGE)
    def _(h):
        cbase = pl.multiple_of(tile_base + h * quarter, 8)
        pltpu.sync_copy(src_hbm.at[pl.ds(cbase, QMAX), :], sbuf)
        pltpu.sync_copy(dst_hbm.at[pl.ds(cbase, QMAX), :], dbuf)
        for b in range(NB):
            pltpu.async_copy(t_hbm.at[sbuf.at[b]], rows_v.at[b], gsem.at[b])

        @pl.loop(0, quarter // NB - 1)
        def _(g):
            for b in range(NB):
                i = g * NB + b
                pltpu.make_async_copy(t_hbm.at[sbuf.at[i]], rows_v.at[b],
                                      gsem.at[b]).wait()
                pltpu.sync_copy(rows_v.at[b], acc_s.at[dbuf.at[i]], add=True)
                pltpu.async_copy(t_hbm.at[sbuf.at[i + NB]], rows_v.at[b],
                                 gsem.at[b])

        for b in range(NB):
            i = quarter - NB + b
            pltpu.make_async_copy(t_hbm.at[sbuf.at[i]], rows_v.at[b],
                                  gsem.at[b]).wait()
            pltpu.sync_copy(rows_v.at[b], acc_s.at[dbuf.at[i]], add=True)

    plsc.subcore_barrier()
    pltpu.sync_copy(acc_s.at[rows], out_hbm.at[c, rows])


R = 512
GRID = NPAD // R


def _norms(deg_blk):
    out_deg = deg_blk[:, 0:1] + deg_blk[:, 2:3]
    in_deg = deg_blk[:, 1:2] + deg_blk[:, 3:4]
    out_norm = lax.rsqrt(jnp.maximum(out_deg, 1.0))
    in_norm = lax.rsqrt(jnp.maximum(in_deg, 1.0))
    return out_norm, in_norm


def _layer_norm_relu(x, g, be):
    mu = jnp.mean(x, axis=-1, keepdims=True)
    var = jnp.mean((x - mu) ** 2, axis=-1, keepdims=True)
    return jnp.maximum((x - mu) * lax.rsqrt(var + EPS) * g + be, 0.0)


def _tc1_body(f_ref, deg_ref, w_ref, t_ref):
    out_norm, _ = _norms(deg_ref[...])
    t_ref[...] = jnp.dot(f_ref[...] * out_norm, w_ref[...],
                         preferred_element_type=jnp.float32)


def _tc2_body(agg_ref, deg_ref, b_ref, g_ref, be_ref, w_ref, t_ref):
    out_norm, in_norm = _norms(deg_ref[...])
    x = (agg_ref[0] + agg_ref[1]) * in_norm + b_ref[...]
    h = _layer_norm_relu(x, g_ref[...], be_ref[...])
    t_ref[...] = jnp.dot(h * out_norm, w_ref[...],
                         preferred_element_type=jnp.float32)


def _tc3_body(agg_ref, deg_ref, b_ref, g_ref, be_ref, o_ref):
    _, in_norm = _norms(deg_ref[...])
    x = (agg_ref[0] + agg_ref[1]) * in_norm + b_ref[...]
    o_ref[...] = _layer_norm_relu(x, g_ref[...], be_ref[...])


_row_spec = pl.BlockSpec((R, D), lambda i: (i, 0))
_deg_spec = pl.BlockSpec((R, 4), lambda i: (i, 0))
_agg_spec = pl.BlockSpec((NC, R, D), lambda i: (0, i, 0))
_w_spec = pl.BlockSpec((D, D), lambda i: (0, 0))
_vec_spec = pl.BlockSpec((1, D), lambda i: (0, 0))

_tc1 = pl.pallas_call(
    _tc1_body,
    grid=(GRID,),
    in_specs=[_row_spec, _deg_spec, _w_spec],
    out_specs=_row_spec,
    out_shape=jax.ShapeDtypeStruct((NPAD, D), jnp.float32),
)

_tc2 = pl.pallas_call(
    _tc2_body,
    grid=(GRID,),
    in_specs=[_agg_spec, _deg_spec, _vec_spec, _vec_spec, _vec_spec, _w_spec],
    out_specs=_row_spec,
    out_shape=jax.ShapeDtypeStruct((NPAD, D), jnp.float32),
)

_tc3 = pl.pallas_call(
    _tc3_body,
    grid=(GRID,),
    in_specs=[_agg_spec, _deg_spec, _vec_spec, _vec_spec, _vec_spec],
    out_specs=_row_spec,
    out_shape=jax.ShapeDtypeStruct((NPAD, D), jnp.float32),
)


def kernel(features, edge_index, W1, b1, g1, be1, W2, b2, g2, be2):
    src = edge_index[0]
    dst = edge_index[1]
    pad = jnp.full((EPAD + QMAX * K - E,), N, dtype=jnp.int32)
    src2d = jnp.concatenate([src, pad]).reshape(EPAD // K + QMAX, K)
    dst2d = jnp.concatenate([dst, pad]).reshape(EPAD // K + QMAX, K)
    fpad = jnp.pad(features, ((0, NPAD - N), (0, 0)))
    zrow = jnp.zeros((NPAD,), jnp.float32)
    zmat = jnp.zeros((NPAD, D), jnp.float32)

    degp = _sc_degrees(src2d, dst2d, zrow)
    degs = jnp.moveaxis(degp.reshape(4, NPAD), 0, 1)

    t1 = _tc1(fpad, degs, W1)
    agg1 = _sc_aggregate(t1, src2d, dst2d, zmat)
    t2 = _tc2(agg1, degs, b1.reshape(1, D), g1.reshape(1, D),
              be1.reshape(1, D), W2)
    agg2 = _sc_aggregate(t2, src2d, dst2d, zmat)
    out = _tc3(agg2, degs, b2.reshape(1, D), g2.reshape(1, D),
               be2.reshape(1, D))
    return out[:N]


# speedup vs baseline: 3.1027x; 2.7770x over previous
import functools

import jax
import jax.numpy as jnp
from jax import lax
from jax.experimental import pallas as pl
from jax.experimental.pallas import tpu as pltpu
from jax.experimental.pallas import tpu_sc as plsc

N = 10000
E = 320000
D = 128
EPS = 1e-5

NC = 2
NS = 16
NW = NC * NS
K = 128
CHUNKS = 80
EPAD = NW * CHUNKS * K
NPAD = 10240
STRIPE = NPAD // NS

_mesh = plsc.VectorSubcoreMesh(core_axis_name="c", subcore_axis_name="s",
                               num_cores=NC, num_subcores=NS)


@functools.partial(
    pl.kernel,
    out_type=jax.ShapeDtypeStruct((NC, 2, NPAD), jnp.float32),
    mesh=_mesh,
    scratch_types=[
        pltpu.VMEM((CHUNKS, K), jnp.int32),
        pltpu.VMEM((CHUNKS, K), jnp.int32),
        pltpu.VMEM((K,), jnp.float32),
        pltpu.VMEM_SHARED((NPAD,), jnp.float32),
        pltpu.VMEM_SHARED((NPAD,), jnp.float32),
    ],
)
def _sc_degrees(src_hbm, dst_hbm, zrow_hbm, out_hbm, sbuf, dbuf, ones_v,
                outd_s, ind_s):
    c = lax.axis_index("c")
    s = lax.axis_index("s")
    wid = c * NS + s
    for i in range(K // 16):
        ones_v[pl.ds(i * 16, 16)] = jnp.ones((16,), jnp.float32)
    rows = pl.ds(s * STRIPE, STRIPE)
    pltpu.sync_copy(zrow_hbm.at[pl.ds(0, STRIPE)], outd_s.at[rows])
    pltpu.sync_copy(zrow_hbm.at[pl.ds(0, STRIPE)], ind_s.at[rows])
    pltpu.sync_copy(src_hbm.at[pl.ds(wid * CHUNKS, CHUNKS), :], sbuf)
    pltpu.sync_copy(dst_hbm.at[pl.ds(wid * CHUNKS, CHUNKS), :], dbuf)
    plsc.subcore_barrier()

    @pl.loop(0, CHUNKS)
    def _(i):
        pltpu.sync_copy(ones_v, outd_s.at[sbuf.at[i]], add=True)
        pltpu.sync_copy(ones_v, ind_s.at[dbuf.at[i]], add=True)

    plsc.subcore_barrier()
    pltpu.sync_copy(outd_s.at[rows], out_hbm.at[c, 0, rows])
    pltpu.sync_copy(ind_s.at[rows], out_hbm.at[c, 1, rows])


NB = 2
CH0 = 80
CH1 = 80
NSTAGE = 2
QMAX = max(CH0, CH1) // NSTAGE


@functools.partial(
    pl.kernel,
    out_type=jax.ShapeDtypeStruct((NC, NPAD, D), jnp.float32),
    mesh=_mesh,
    scratch_types=[
        pltpu.VMEM((QMAX, K), jnp.int32),
        pltpu.VMEM((QMAX, K), jnp.int32),
        pltpu.VMEM((NB, K, D), jnp.float32),
        pltpu.VMEM_SHARED((NPAD, D), jnp.float32),
        pltpu.SemaphoreType.DMA((NB,)),
    ],
)
def _sc_aggregate(t_hbm, src_hbm, dst_hbm, zmat_hbm, out_hbm, sbuf, dbuf,
                  rows_v, acc_s, gsem):
    c = lax.axis_index("c")
    s = lax.axis_index("s")
    rows = pl.ds(s * STRIPE, STRIPE)
    pltpu.sync_copy(zmat_hbm.at[rows], acc_s.at[rows])
    plsc.subcore_barrier()

    my_ch = jnp.where(c == 0, CH0, CH1)
    quarter = my_ch // NSTAGE
    tile_base = jnp.where(c == 0, s * CH0, NS * CH0 + s * CH1)

    @pl.loop(0, NSTAGE)
    def _(h):
        cbase = pl.multiple_of(tile_base + h * quarter, 8)
        pltpu.sync_copy(src_hbm.at[pl.ds(cbase, QMAX), :], sbuf)
        pltpu.sync_copy(dst_hbm.at[pl.ds(cbase, QMAX), :], dbuf)
        for b in range(NB):
            pltpu.async_copy(t_hbm.at[sbuf.at[b]], rows_v.at[b], gsem.at[b])

        @pl.loop(0, quarter // NB - 1)
        def _(g):
            for b in range(NB):
                i = g * NB + b
                pltpu.make_async_copy(t_hbm.at[sbuf.at[i]], rows_v.at[b],
                                      gsem.at[b]).wait()
                pltpu.sync_copy(rows_v.at[b], acc_s.at[dbuf.at[i]], add=True)
                pltpu.async_copy(t_hbm.at[sbuf.at[i + NB]], rows_v.at[b],
                                 gsem.at[b])

        for b in range(NB):
            i = quarter - NB + b
            pltpu.make_async_copy(t_hbm.at[sbuf.at[i]], rows_v.at[b],
                                  gsem.at[b]).wait()
            pltpu.sync_copy(rows_v.at[b], acc_s.at[dbuf.at[i]], add=True)

    plsc.subcore_barrier()
    pltpu.sync_copy(acc_s.at[rows], out_hbm.at[c, rows])


R = 512
GRID = NPAD // R


def _norms(deg_blk):
    out_deg = deg_blk[:, 0:1] + deg_blk[:, 2:3]
    in_deg = deg_blk[:, 1:2] + deg_blk[:, 3:4]
    out_norm = lax.rsqrt(jnp.maximum(out_deg, 1.0))
    in_norm = lax.rsqrt(jnp.maximum(in_deg, 1.0))
    return out_norm, in_norm


def _layer_norm_relu(x, g, be):
    mu = jnp.mean(x, axis=-1, keepdims=True)
    var = jnp.mean((x - mu) ** 2, axis=-1, keepdims=True)
    return jnp.maximum((x - mu) * lax.rsqrt(var + EPS) * g + be, 0.0)


def _tc1_body(f_ref, deg_ref, w_ref, t_ref):
    out_norm, _ = _norms(deg_ref[...])
    t_ref[...] = jnp.dot(f_ref[...] * out_norm, w_ref[...],
                         preferred_element_type=jnp.float32)


def _tc2_body(agg_ref, deg_ref, b_ref, g_ref, be_ref, w_ref, t_ref):
    out_norm, in_norm = _norms(deg_ref[...])
    x = (agg_ref[0] + agg_ref[1]) * in_norm + b_ref[...]
    h = _layer_norm_relu(x, g_ref[...], be_ref[...])
    t_ref[...] = jnp.dot(h * out_norm, w_ref[...],
                         preferred_element_type=jnp.float32)


def _tc3_body(agg_ref, deg_ref, b_ref, g_ref, be_ref, o_ref):
    _, in_norm = _norms(deg_ref[...])
    x = (agg_ref[0] + agg_ref[1]) * in_norm + b_ref[...]
    o_ref[...] = _layer_norm_relu(x, g_ref[...], be_ref[...])


_row_spec = pl.BlockSpec((R, D), lambda i: (i, 0))
_deg_spec = pl.BlockSpec((R, 4), lambda i: (i, 0))
_agg_spec = pl.BlockSpec((NC, R, D), lambda i: (0, i, 0))
_w_spec = pl.BlockSpec((D, D), lambda i: (0, 0))
_vec_spec = pl.BlockSpec((1, D), lambda i: (0, 0))

_tc1 = pl.pallas_call(
    _tc1_body,
    grid=(GRID,),
    in_specs=[_row_spec, _deg_spec, _w_spec],
    out_specs=_row_spec,
    out_shape=jax.ShapeDtypeStruct((NPAD, D), jnp.float32),
)

_tc2 = pl.pallas_call(
    _tc2_body,
    grid=(GRID,),
    in_specs=[_agg_spec, _deg_spec, _vec_spec, _vec_spec, _vec_spec, _w_spec],
    out_specs=_row_spec,
    out_shape=jax.ShapeDtypeStruct((NPAD, D), jnp.float32),
)

_tc3 = pl.pallas_call(
    _tc3_body,
    grid=(GRID,),
    in_specs=[_agg_spec, _deg_spec, _vec_spec, _vec_spec, _vec_spec],
    out_specs=_row_spec,
    out_shape=jax.ShapeDtypeStruct((NPAD, D), jnp.float32),
)


def kernel(features, edge_index, W1, b1, g1, be1, W2, b2, g2, be2):
    src = edge_index[0]
    dst = edge_index[1]
    padn = EPAD + QMAX * K - E
    pad = N + (jnp.arange(padn, dtype=jnp.int32) % (NPAD - N))
    src2d = jnp.concatenate([src, pad]).reshape(EPAD // K + QMAX, K)
    dst2d = jnp.concatenate([dst, pad]).reshape(EPAD // K + QMAX, K)
    fpad = jnp.pad(features, ((0, NPAD - N), (0, 0)))
    zrow = jnp.zeros((NPAD,), jnp.float32)
    zmat = jnp.zeros((NPAD, D), jnp.float32)

    degp = _sc_degrees(src2d, dst2d, zrow)
    degs = jnp.moveaxis(degp.reshape(4, NPAD), 0, 1)

    t1 = _tc1(fpad, degs, W1)
    agg1 = _sc_aggregate(t1, src2d, dst2d, zmat)
    t2 = _tc2(agg1, degs, b1.reshape(1, D), g1.reshape(1, D),
              be1.reshape(1, D), W2)
    agg2 = _sc_aggregate(t2, src2d, dst2d, zmat)
    out = _tc3(agg2, degs, b2.reshape(1, D), g2.reshape(1, D),
               be2.reshape(1, D))
    return out[:N]


# speedup vs baseline: 3.3054x; 1.0653x over previous
import functools

import numpy as np
import jax
import jax.numpy as jnp
from jax import lax
from jax.experimental import pallas as pl
from jax.experimental.pallas import tpu as pltpu
from jax.experimental.pallas import tpu_sc as plsc

N = 10000
E = 320000
D = 128
EPS = 1e-5

NC = 2
NS = 16
NW = NC * NS
K = 128
CHUNKS = 80
EPAD = NW * CHUNKS * K
NPAD = 10240
STRIPE = NPAD // NS
NB = 2
NSTAGE = 2
QMAX = CHUNKS // NSTAGE
CROWS = EPAD // K + QMAX

_PADV = (np.arange(CROWS * K - E, dtype=np.int32) % (NPAD - N)) + N
_EPAD_CONST = np.stack([_PADV, _PADV])

_mesh = plsc.VectorSubcoreMesh(core_axis_name="c", subcore_axis_name="s",
                               num_cores=NC, num_subcores=NS)


@functools.partial(
    pl.kernel,
    out_type=jax.ShapeDtypeStruct((NC, 2, NPAD), jnp.float32),
    mesh=_mesh,
    scratch_types=[
        pltpu.VMEM((CHUNKS, K), jnp.int32),
        pltpu.VMEM((CHUNKS, K), jnp.int32),
        pltpu.VMEM((K,), jnp.float32),
        pltpu.VMEM_SHARED((NPAD,), jnp.float32),
        pltpu.VMEM_SHARED((NPAD,), jnp.float32),
    ],
)
def _sc_degrees(src_hbm, dst_hbm, zrow_hbm, out_hbm, sbuf, dbuf, ones_v,
                outd_s, ind_s):
    c = lax.axis_index("c")
    s = lax.axis_index("s")
    wid = c * NS + s
    for i in range(K // 16):
        ones_v[pl.ds(i * 16, 16)] = jnp.ones((16,), jnp.float32)
    rows = pl.ds(s * STRIPE, STRIPE)
    pltpu.sync_copy(zrow_hbm.at[pl.ds(0, STRIPE)], outd_s.at[rows])
    pltpu.sync_copy(zrow_hbm.at[pl.ds(0, STRIPE)], ind_s.at[rows])
    pltpu.sync_copy(src_hbm.at[pl.ds(wid * CHUNKS, CHUNKS), :], sbuf)
    pltpu.sync_copy(dst_hbm.at[pl.ds(wid * CHUNKS, CHUNKS), :], dbuf)
    plsc.subcore_barrier()

    @pl.loop(0, CHUNKS)
    def _(i):
        pltpu.sync_copy(ones_v, outd_s.at[sbuf.at[i]], add=True)
        pltpu.sync_copy(ones_v, ind_s.at[dbuf.at[i]], add=True)

    plsc.subcore_barrier()
    pltpu.sync_copy(outd_s.at[rows], out_hbm.at[c, 0, rows])
    pltpu.sync_copy(ind_s.at[rows], out_hbm.at[c, 1, rows])


@functools.partial(
    pl.kernel,
    out_type=jax.ShapeDtypeStruct((NC, NPAD, D), jnp.float32),
    mesh=_mesh,
    scratch_types=[
        pltpu.VMEM((QMAX, K), jnp.int32),
        pltpu.VMEM((QMAX, K), jnp.int32),
        pltpu.VMEM((NB, K, D), jnp.float32),
        pltpu.VMEM_SHARED((NPAD, D), jnp.float32),
        pltpu.SemaphoreType.DMA((NB,)),
    ],
)
def _sc_aggregate(t_hbm, src_hbm, dst_hbm, zmat_hbm, out_hbm, sbuf, dbuf,
                  rows_v, acc_s, gsem):
    c = lax.axis_index("c")
    s = lax.axis_index("s")
    wid = c * NS + s
    rows = pl.ds(s * STRIPE, STRIPE)
    pltpu.sync_copy(zmat_hbm.at[rows], acc_s.at[rows])
    plsc.subcore_barrier()

    @pl.loop(0, NSTAGE)
    def _(h):
        cbase = pl.multiple_of(wid * CHUNKS + h * QMAX, 8)
        pltpu.sync_copy(src_hbm.at[pl.ds(cbase, QMAX), :], sbuf)
        pltpu.sync_copy(dst_hbm.at[pl.ds(cbase, QMAX), :], dbuf)
        for b in range(NB):
            pltpu.async_copy(t_hbm.at[sbuf.at[b]], rows_v.at[b], gsem.at[b])

        @pl.loop(0, QMAX // NB - 1)
        def _(g):
            for b in range(NB):
                i = g * NB + b
                pltpu.make_async_copy(t_hbm.at[sbuf.at[i]], rows_v.at[b],
                                      gsem.at[b]).wait()
                pltpu.sync_copy(rows_v.at[b], acc_s.at[dbuf.at[i]], add=True)
                pltpu.async_copy(t_hbm.at[sbuf.at[i + NB]], rows_v.at[b],
                                 gsem.at[b])

        for b in range(NB):
            i = QMAX - NB + b
            pltpu.make_async_copy(t_hbm.at[sbuf.at[i]], rows_v.at[b],
                                  gsem.at[b]).wait()
            pltpu.sync_copy(rows_v.at[b], acc_s.at[dbuf.at[i]], add=True)

    plsc.subcore_barrier()
    pltpu.sync_copy(acc_s.at[rows], out_hbm.at[c, rows])


R = 1024
GRID = NPAD // R
RO = 1000
GRIDO = N // RO


def _norms(deg_blk):
    out_deg = deg_blk[:, 0:1] + deg_blk[:, 2:3]
    in_deg = deg_blk[:, 1:2] + deg_blk[:, 3:4]
    out_norm = lax.rsqrt(jnp.maximum(out_deg, 1.0))
    in_norm = lax.rsqrt(jnp.maximum(in_deg, 1.0))
    return out_norm, in_norm


def _layer_norm_relu(x, g, be):
    mu = jnp.mean(x, axis=-1, keepdims=True)
    var = jnp.mean((x - mu) ** 2, axis=-1, keepdims=True)
    return jnp.maximum((x - mu) * lax.rsqrt(var + EPS) * g + be, 0.0)


def _tc1_body(f_ref, deg_ref, w_ref, t_ref):
    out_norm, _ = _norms(deg_ref[...])
    t_ref[...] = jnp.dot(f_ref[...] * out_norm, w_ref[...],
                         preferred_element_type=jnp.float32)


def _tc2_body(agg_ref, deg_ref, b_ref, g_ref, be_ref, w_ref, t_ref):
    out_norm, in_norm = _norms(deg_ref[...])
    x = (agg_ref[0] + agg_ref[1]) * in_norm + b_ref[...]
    h = _layer_norm_relu(x, g_ref[...], be_ref[...])
    t_ref[...] = jnp.dot(h * out_norm, w_ref[...],
                         preferred_element_type=jnp.float32)


def _tc3_body(agg_ref, deg_ref, b_ref, g_ref, be_ref, o_ref):
    _, in_norm = _norms(deg_ref[...])
    x = (agg_ref[0] + agg_ref[1]) * in_norm + b_ref[...]
    o_ref[...] = _layer_norm_relu(x, g_ref[...], be_ref[...])


def _specs(r):
    return (pl.BlockSpec((r, D), lambda i: (i, 0)),
            pl.BlockSpec((r, 4), lambda i: (i, 0)),
            pl.BlockSpec((NC, r, D), lambda i: (0, i, 0)))


_row_spec, _deg_spec, _agg_spec = _specs(R)
_rowo_spec, _dego_spec, _aggo_spec = _specs(RO)
_w_spec = pl.BlockSpec((D, D), lambda i: (0, 0))
_vec_spec = pl.BlockSpec((1, D), lambda i: (0, 0))

_tc1 = pl.pallas_call(
    _tc1_body,
    grid=(GRID,),
    in_specs=[_row_spec, _deg_spec, _w_spec],
    out_specs=_row_spec,
    out_shape=jax.ShapeDtypeStruct((NPAD, D), jnp.float32),
)

_tc2 = pl.pallas_call(
    _tc2_body,
    grid=(GRID,),
    in_specs=[_agg_spec, _deg_spec, _vec_spec, _vec_spec, _vec_spec, _w_spec],
    out_specs=_row_spec,
    out_shape=jax.ShapeDtypeStruct((NPAD, D), jnp.float32),
)

_tc3 = pl.pallas_call(
    _tc3_body,
    grid=(GRIDO,),
    in_specs=[_aggo_spec, _dego_spec, _vec_spec, _vec_spec, _vec_spec],
    out_specs=_rowo_spec,
    out_shape=jax.ShapeDtypeStruct((N, D), jnp.float32),
)


def kernel(features, edge_index, W1, b1, g1, be1, W2, b2, g2, be2):
    padc = jnp.asarray(_EPAD_CONST)
    src2d = jnp.concatenate([edge_index[0], padc[0]]).reshape(CROWS, K)
    dst2d = jnp.concatenate([edge_index[1], padc[1]]).reshape(CROWS, K)
    fpad = jnp.pad(features, ((0, NPAD - N), (0, 0)))
    zrow = jnp.zeros((NPAD,), jnp.float32)
    zmat = jnp.zeros((NPAD, D), jnp.float32)

    degp = _sc_degrees(src2d, dst2d, zrow)
    degs = jnp.moveaxis(degp.reshape(4, NPAD), 0, 1)

    t1 = _tc1(fpad, degs, W1)
    agg1 = _sc_aggregate(t1, src2d, dst2d, zmat)
    t2 = _tc2(agg1, degs, b1.reshape(1, D), g1.reshape(1, D),
              be1.reshape(1, D), W2)
    agg2 = _sc_aggregate(t2, src2d, dst2d, zmat)
    return _tc3(agg2, degs, b2.reshape(1, D), g2.reshape(1, D),
                be2.reshape(1, D))


# speedup vs baseline: 3.5512x; 1.0744x over previous
import functools

import numpy as np
import jax
import jax.numpy as jnp
from jax import lax
from jax.experimental import pallas as pl
from jax.experimental.pallas import tpu as pltpu
from jax.experimental.pallas import tpu_sc as plsc

N = 10000
E = 320000
D = 128
EPS = 1e-5

NC = 2
NS = 16
NW = NC * NS
K = 128
CHUNKS = 80
EPAD = NW * CHUNKS * K
NPAD = 10240
STRIPE = NPAD // NS
NB = 2
NSTAGE = 2
QMAX = CHUNKS // NSTAGE
CROWS = EPAD // K + QMAX

_PADV = (np.arange(CROWS * K - E, dtype=np.int32) % (NPAD - N)) + N
_EPAD_CONST = np.stack([_PADV, _PADV])

_mesh = plsc.VectorSubcoreMesh(core_axis_name="c", subcore_axis_name="s",
                               num_cores=NC, num_subcores=NS)


WAVE = 16


@functools.partial(
    pl.kernel,
    out_type=jax.ShapeDtypeStruct((NC, 2, NPAD), jnp.float32),
    mesh=_mesh,
    scratch_types=[
        pltpu.VMEM((CHUNKS, K), jnp.int32),
        pltpu.VMEM((CHUNKS, K), jnp.int32),
        pltpu.VMEM((K,), jnp.float32),
        pltpu.VMEM((STRIPE,), jnp.float32),
        pltpu.VMEM_SHARED((NPAD,), jnp.float32),
        pltpu.VMEM_SHARED((NPAD,), jnp.float32),
        pltpu.SemaphoreType.DMA,
    ],
)
def _sc_degrees(src_hbm, dst_hbm, out_hbm, sbuf, dbuf, ones_v, zero_v,
                outd_s, ind_s, dsem):
    c = lax.axis_index("c")
    s = lax.axis_index("s")
    wid = c * NS + s
    for i in range(K // 16):
        ones_v[pl.ds(i * 16, 16)] = jnp.ones((16,), jnp.float32)

    @pl.loop(0, STRIPE // 16)
    def _(i):
        zero_v[pl.ds(i * 16, 16)] = jnp.zeros((16,), jnp.float32)

    rows = pl.ds(s * STRIPE, STRIPE)
    pltpu.sync_copy(zero_v, outd_s.at[rows])
    pltpu.sync_copy(zero_v, ind_s.at[rows])
    pltpu.sync_copy(src_hbm.at[pl.ds(wid * CHUNKS, CHUNKS), :], sbuf)
    pltpu.sync_copy(dst_hbm.at[pl.ds(wid * CHUNKS, CHUNKS), :], dbuf)
    plsc.subcore_barrier()

    @pl.loop(0, CHUNKS // WAVE)
    def _(w):
        for j in range(WAVE):
            i = w * WAVE + j
            pltpu.async_copy(ones_v, outd_s.at[sbuf.at[i]], dsem, add=True)
            pltpu.async_copy(ones_v, ind_s.at[dbuf.at[i]], dsem, add=True)
        for j in range(WAVE):
            i = w * WAVE + j
            pltpu.make_async_copy(ones_v, outd_s.at[sbuf.at[i]], dsem).wait()
            pltpu.make_async_copy(ones_v, ind_s.at[dbuf.at[i]], dsem).wait()

    plsc.subcore_barrier()
    pltpu.sync_copy(outd_s.at[rows], out_hbm.at[c, 0, rows])
    pltpu.sync_copy(ind_s.at[rows], out_hbm.at[c, 1, rows])


@functools.partial(
    pl.kernel,
    out_type=jax.ShapeDtypeStruct((NC, NPAD, D), jnp.float32),
    mesh=_mesh,
    scratch_types=[
        pltpu.VMEM((QMAX, K), jnp.int32),
        pltpu.VMEM((QMAX, K), jnp.int32),
        pltpu.VMEM((NB, K, D), jnp.float32),
        pltpu.VMEM_SHARED((NPAD, D), jnp.float32),
        pltpu.SemaphoreType.DMA((NB,)),
    ],
)
def _sc_aggregate(t_hbm, src_hbm, dst_hbm, out_hbm, sbuf, dbuf,
                  rows_v, acc_s, gsem):
    c = lax.axis_index("c")
    s = lax.axis_index("s")
    wid = c * NS + s
    rows = pl.ds(s * STRIPE, STRIPE)

    @pl.loop(0, K)
    def _(r):
        for j in range(D // 16):
            rows_v[0, r, pl.ds(j * 16, 16)] = jnp.zeros((16,), jnp.float32)

    @pl.loop(0, STRIPE // K)
    def _(j):
        pltpu.sync_copy(rows_v.at[0],
                        acc_s.at[pl.ds(s * STRIPE + j * K, K)])
    plsc.subcore_barrier()

    @pl.loop(0, NSTAGE)
    def _(h):
        cbase = pl.multiple_of(wid * CHUNKS + h * QMAX, 8)
        pltpu.sync_copy(src_hbm.at[pl.ds(cbase, QMAX), :], sbuf)
        pltpu.sync_copy(dst_hbm.at[pl.ds(cbase, QMAX), :], dbuf)
        for b in range(NB):
            pltpu.async_copy(t_hbm.at[sbuf.at[b]], rows_v.at[b], gsem.at[b])

        @pl.loop(0, QMAX // NB - 1)
        def _(g):
            for b in range(NB):
                i = g * NB + b
                pltpu.make_async_copy(t_hbm.at[sbuf.at[i]], rows_v.at[b],
                                      gsem.at[b]).wait()
                pltpu.sync_copy(rows_v.at[b], acc_s.at[dbuf.at[i]], add=True)
                pltpu.async_copy(t_hbm.at[sbuf.at[i + NB]], rows_v.at[b],
                                 gsem.at[b])

        for b in range(NB):
            i = QMAX - NB + b
            pltpu.make_async_copy(t_hbm.at[sbuf.at[i]], rows_v.at[b],
                                  gsem.at[b]).wait()
            pltpu.sync_copy(rows_v.at[b], acc_s.at[dbuf.at[i]], add=True)

    plsc.subcore_barrier()
    pltpu.sync_copy(acc_s.at[rows], out_hbm.at[c, rows])


R = 1024
GRID = NPAD // R
RO = 1000
GRIDO = N // RO


def _norms(deg_blk):
    out_deg = deg_blk[:, 0:1] + deg_blk[:, 2:3]
    in_deg = deg_blk[:, 1:2] + deg_blk[:, 3:4]
    out_norm = lax.rsqrt(jnp.maximum(out_deg, 1.0))
    in_norm = lax.rsqrt(jnp.maximum(in_deg, 1.0))
    return out_norm, in_norm


def _layer_norm_relu(x, g, be):
    mu = jnp.mean(x, axis=-1, keepdims=True)
    var = jnp.mean((x - mu) ** 2, axis=-1, keepdims=True)
    return jnp.maximum((x - mu) * lax.rsqrt(var + EPS) * g + be, 0.0)


def _tc1_body(f_ref, deg_ref, w_ref, t_ref):
    out_norm, _ = _norms(deg_ref[...])
    t_ref[...] = jnp.dot(f_ref[...] * out_norm, w_ref[...],
                         preferred_element_type=jnp.float32)


def _tc2_body(agg_ref, deg_ref, b_ref, g_ref, be_ref, w_ref, t_ref):
    out_norm, in_norm = _norms(deg_ref[...])
    x = (agg_ref[0] + agg_ref[1]) * in_norm + b_ref[...]
    h = _layer_norm_relu(x, g_ref[...], be_ref[...])
    t_ref[...] = jnp.dot(h * out_norm, w_ref[...],
                         preferred_element_type=jnp.float32)


def _tc3_body(agg_ref, deg_ref, b_ref, g_ref, be_ref, o_ref):
    _, in_norm = _norms(deg_ref[...])
    x = (agg_ref[0] + agg_ref[1]) * in_norm + b_ref[...]
    o_ref[...] = _layer_norm_relu(x, g_ref[...], be_ref[...])


def _specs(r):
    return (pl.BlockSpec((r, D), lambda i: (i, 0)),
            pl.BlockSpec((r, 4), lambda i: (i, 0)),
            pl.BlockSpec((NC, r, D), lambda i: (0, i, 0)))


_row_spec, _deg_spec, _agg_spec = _specs(R)
_rowo_spec, _dego_spec, _aggo_spec = _specs(RO)
_w_spec = pl.BlockSpec((D, D), lambda i: (0, 0))
_vec_spec = pl.BlockSpec((1, D), lambda i: (0, 0))

_tc1 = pl.pallas_call(
    _tc1_body,
    grid=(GRID,),
    in_specs=[_row_spec, _deg_spec, _w_spec],
    out_specs=_row_spec,
    out_shape=jax.ShapeDtypeStruct((NPAD, D), jnp.float32),
)

_tc2 = pl.pallas_call(
    _tc2_body,
    grid=(GRID,),
    in_specs=[_agg_spec, _deg_spec, _vec_spec, _vec_spec, _vec_spec, _w_spec],
    out_specs=_row_spec,
    out_shape=jax.ShapeDtypeStruct((NPAD, D), jnp.float32),
)

_tc3 = pl.pallas_call(
    _tc3_body,
    grid=(GRIDO,),
    in_specs=[_aggo_spec, _dego_spec, _vec_spec, _vec_spec, _vec_spec],
    out_specs=_rowo_spec,
    out_shape=jax.ShapeDtypeStruct((N, D), jnp.float32),
)


def kernel(features, edge_index, W1, b1, g1, be1, W2, b2, g2, be2):
    padc = jnp.asarray(_EPAD_CONST)
    src2d = jnp.concatenate([edge_index[0], padc[0]]).reshape(CROWS, K)
    dst2d = jnp.concatenate([edge_index[1], padc[1]]).reshape(CROWS, K)

    degp = _sc_degrees(src2d, dst2d)
    degs = jnp.moveaxis(degp.reshape(4, NPAD), 0, 1)

    t1 = _tc1(features, degs, W1)
    agg1 = _sc_aggregate(t1, src2d, dst2d)
    t2 = _tc2(agg1, degs, b1.reshape(1, D), g1.reshape(1, D),
              be1.reshape(1, D), W2)
    agg2 = _sc_aggregate(t2, src2d, dst2d)
    return _tc3(agg2, degs, b2.reshape(1, D), g2.reshape(1, D),
                be2.reshape(1, D))
